# baseline (speedup 1.0000x reference)
import dataclasses

import jax
import jax.numpy as jnp
import numpy as np
from jax.experimental import pallas as pl
from jax.experimental.pallas import tpu as pltpu
from jax.experimental.pallas import tpu_sc as plsc

_S = 1024
_N = 50
_NMAX = 64
_D = 64
_K = 100000
_EPS = 0.1
_E_FAC = np.float32(1.0 / np.sqrt(1.0 + _EPS ** 2))
_C_NOISE = np.float32(_EPS / np.sqrt(_D))
_P = 2 * _NMAX + 1
_T = 2 * _N + 1

_NW = 32
_RPAD = 56
_NIP = _RPAD * _S
_CH = 8
_ICH = 4
_NIC = _RPAD // _ICH
_DPW = _D // (_NW // 2)
_B = 128


def _sc_gather_t(cls_t, lab_t, idxc, idxl):
    mesh = plsc.VectorSubcoreMesh(core_axis_name="c", subcore_axis_name="s")
    cp = pltpu.CompilerParams()
    if "needs_layout_passes" in pltpu.CompilerParams.__dataclass_fields__:
        cp = dataclasses.replace(cp, needs_layout_passes=False)

    @pl.kernel(
        out_type=(
            jax.ShapeDtypeStruct((_D, _RPAD, _S), jnp.float32),
            jax.ShapeDtypeStruct((_D, _RPAD, _S), jnp.float32),
        ),
        mesh=mesh,
        compiler_params=cp,
        scratch_types=[
            pltpu.VMEM((_K,), jnp.float32),
            pltpu.VMEM((_ICH * _S,), jnp.int32),
            pltpu.VMEM((_ICH * _S,), jnp.int32),
            pltpu.VMEM((_CH, _S), jnp.float32),
            pltpu.VMEM((_CH, _S), jnp.float32),
            pltpu.SemaphoreType.DMA,
            pltpu.SemaphoreType.DMA,
            pltpu.SemaphoreType.DMA,
        ],
    )
    def k_fn(cls_hbm, lab_hbm, ic_hbm, il_hbm, oc_hbm, ol_hbm,
             row_v, idxa_v, idxb_v, outa_v, outb_v, rsem, isem, osem):
        wid = jax.lax.axis_index("s") * 2 + jax.lax.axis_index("c")
        idxs = (idxa_v, idxb_v)
        outs = (outa_v, outb_v)

        def task(tab_hbm, i_hbm, o_hbm, d):
            rh = pltpu.async_copy(tab_hbm.at[d], row_v, rsem)
            ih = [None, None]
            oh = [None, None]
            ih[0] = pltpu.async_copy(i_hbm.at[pl.ds(0, _ICH * _S)],
                                     idxa_v, isem)
            rh.wait()
            for c in range(_NIC):
                b = c % 2
                ih[b].wait()
                if c + 1 < _NIC:
                    ih[1 - b] = pltpu.async_copy(
                        i_hbm.at[pl.ds((c + 1) * _ICH * _S, _ICH * _S)],
                        idxs[1 - b], isem)
                half = c % 2
                ob = (c // 2) % 2
                o_v = outs[ob]
                if half == 0 and oh[ob] is not None:
                    oh[ob].wait()
                i_v = idxs[b]

                @pl.loop(0, _ICH * _S // 16, step=8)
                def _(j0):
                    for u in range(8):
                        j = j0 + u
                        idx16 = i_v[pl.ds(j * 16, 16)]
                        vals = plsc.load_gather(row_v, [idx16])
                        jj = j + half * (_ICH * _S // 16)
                        r = jj // (_S // 16)
                        col = (jj % (_S // 16)) * 16
                        o_v[r, pl.ds(col, 16)] = vals

                if half == 1:
                    oh[ob] = pltpu.async_copy(
                        o_v, o_hbm.at[d, pl.ds((c // 2) * _CH, _CH)], osem)
            for h in oh:
                if h is not None:
                    h.wait()

        half = wid % (_NW // 2)

        @pl.when(wid < _NW // 2)
        def _():
            for m in range(_DPW):
                task(cls_hbm, ic_hbm, oc_hbm, half * _DPW + m)

        @pl.when(wid >= _NW // 2)
        def _():
            for m in range(_DPW):
                task(lab_hbm, il_hbm, ol_hbm, half * _DPW + m)

    return k_fn(cls_t, lab_t, idxc, idxl)


def _assemble_body_t(shift_ref, gc_ref, gl_ref, nc_ref, nq_ref, out_ref):
    gc = gc_ref[...][:, :_N + 1, :]
    gl = gl_ref[...][:, :_N, :]
    nc = jnp.swapaxes(nc_ref[...], 0, 1)
    nq = nq_ref[...]
    ctx = _E_FAC * (gc[:, :_N, :] + _C_NOISE * nc)
    q = _E_FAC * (gc[:, _N, :] + _C_NOISE * nq)
    pair = jnp.stack([ctx, gl], axis=2).reshape(_D, 2 * _N, _B)
    feat = jnp.concatenate([pair, q[:, None, :]], axis=1)
    sh = shift_ref[...].reshape(1, 1, _B)
    j_io = jax.lax.broadcasted_iota(jnp.int32, (_P, _T, _B), 0)
    t_io = jax.lax.broadcasted_iota(jnp.int32, (_P, _T, _B), 1)
    pos = (j_io == t_io + sh).astype(jnp.float32)
    out_ref[pl.ds(0, _P)] = pos
    out_ref[pl.ds(_P, _D)] = feat


def _assemble_t(shifts, gc_t, gl_t, nc_t, nq_t):
    grid = (_S // _B,)
    return pl.pallas_call(
        _assemble_body_t,
        grid=grid,
        in_specs=[
            pl.BlockSpec((_B,), lambda i: (i,)),
            pl.BlockSpec((_D, _RPAD, _B), lambda i: (0, 0, i)),
            pl.BlockSpec((_D, _RPAD, _B), lambda i: (0, 0, i)),
            pl.BlockSpec((_N, _D, _B), lambda i: (0, 0, i)),
            pl.BlockSpec((_D, _B), lambda i: (0, i)),
        ],
        out_specs=pl.BlockSpec((_P + _D, _T, _B), lambda i: (0, 0, i)),
        out_shape=jax.ShapeDtypeStruct((_P + _D, _T, _S), jnp.float32),
    )(shifts, gc_t, gl_t, nc_t, nq_t)


def kernel(example, label, noise_ctx, noise_q, shifts, mus_label, mus_class):
    example = example.astype(jnp.int32)
    label = label.astype(jnp.int32)
    idxc = jnp.pad(example.T.reshape(-1), (0, _NIP - _S * (_N + 1)))
    idxl = jnp.pad(label.T[:_N].reshape(-1), (0, _NIP - _S * _N))
    cls_t = mus_class.T
    lab_t = mus_label.T
    nc_t = noise_ctx.transpose(1, 2, 0)
    nq_t = noise_q.T

    gc_t, gl_t = _sc_gather_t(cls_t, lab_t, idxc, idxl)
    out_t = _assemble_t(shifts.astype(jnp.int32), gc_t, gl_t, nc_t, nq_t)
    return out_t.transpose(2, 1, 0)
